# baseline (speedup 1.0000x reference)
import functools
import math

import jax
import jax.numpy as jnp
from jax import lax
from jax.experimental import pallas as pl
from jax.experimental.pallas import tpu as pltpu
from jax.experimental.pallas import tpu_sc as plsc

N_GENES = 40000
N_DRUGS = 10000
N = N_GENES + N_DRUGS
D = 64
HALF = 32
HOPS = 3
B = 4096
DECAY = 1e-4

NC = 2
NS = 16
CHUNK = 128
SUPER = 1024
SROWS = SUPER // CHUNK
NGB = 4
NSB = 2
SAMP = B // NS
NPAD = 50176
ROWS_PT = NPAD // NS
CP = 112


def _sc_body(epad, tab_h, dst_h, src_h, val_h, tall_h, g_h, p_h, n_h,
             u_o, po_o, ne_o, s_o,
             dstb, srcb, valb, rows, rows1, rows2, rows3, rows4, rows5,
             gidx, pidx, nidx,
             gt, pt, nt, wbuf, acc_sh, sem, semg0, semg1, semg2, semg3,
             sems0, sems1):
    c = lax.axis_index("c")
    s = lax.axis_index("s")
    sbase = s * SAMP
    ept = epad // NS
    n_super = ept // SUPER
    erow0 = s * (ept // CHUNK)

    kinds = ((gidx, gt, u_o, g_h, 0), (pidx, pt, po_o, p_h, N_GENES),
             (nidx, nt, ne_o, n_h, N_GENES))

    for idx, tv, _, src_h_k, off in kinds:
        pltpu.sync_copy(src_h_k.at[pl.ds(sbase, SAMP)], idx)
        if off:
            @plsc.parallel_loop(0, SAMP, 16)
            def _(i):
                idx[pl.ds(i, 16)] = idx[pl.ds(i, 16)] + off
        for j in range(SAMP // CHUNK):
            pltpu.async_copy(
                tall_h.at[idx.at[pl.ds(j * CHUNK, CHUNK)]],
                tv.at[pl.ds(j * CHUNK, CHUNK)], sem).wait()

    def samp_accum(k):
        fk = 1.0 / float(math.factorial(k))
        src_tab = tab_h if k == 0 else s_o
        for idx, tv, out, _, _ in kinds:
            @plsc.parallel_loop(0, SAMP, 16)
            def _(i):
                t = tv[pl.ds(i, 16)]
                tk = jnp.full((16,), fk, jnp.float32)
                for _ in range(k):
                    tk = tk * t
                wbuf[pl.ds(i, 16)] = jnp.exp(-t) * tk

            def sjloop(j, scarry):
                pltpu.async_copy(
                    src_tab.at[c].at[idx.at[pl.ds(j * CHUNK, CHUNK)]],
                    rows, sem).wait()

                @plsc.parallel_loop(0, CHUNK, 16)
                def _(i):
                    wv = wbuf[pl.ds(j * CHUNK + i, 16)]
                    for l in range(16):
                        w = jnp.full((16,), wv[l], jnp.float32)
                        rows[i + l, pl.ds(0, 16)] = rows[i + l, pl.ds(0, 16)] * w
                        rows[i + l, pl.ds(16, 16)] = (
                            rows[i + l, pl.ds(16, 16)] * w)

                pltpu.sync_copy(
                    rows, out.at[c, k, pl.ds(sbase + j * CHUNK, CHUNK)])
                return scarry
            lax.fori_loop(0, SAMP // CHUNK, sjloop, 0)

    def hop(first, acc_sh):
        src_tab = tab_h if first else s_o

        @plsc.parallel_loop(0, CHUNK, 1, unroll=8)
        def _(i):
            z = jnp.zeros((16,), jnp.float32)
            rows[i, pl.ds(0, 16)] = z
            rows[i, pl.ds(16, 16)] = z

        rbase = s * ROWS_PT

        def zcopy(j, carry):
            pltpu.sync_copy(rows.at[pl.ds(0, CP)],
                            acc_sh.at[pl.ds(rbase + j * CP, CP)])
            return carry
        lax.fori_loop(0, ROWS_PT // CP, zcopy, 0)
        plsc.subcore_barrier()

        gbufs = (rows, rows1, rows2, rows3)
        gsems = (semg0, semg1, semg2, semg3)
        sbufs = (rows4, rows5)
        ssems = (sems0, sems1)

        def echunk(sc_i, carry):
            brow = erow0 + sc_i * SROWS
            pltpu.async_copy(dst_h.at[pl.ds(brow, SROWS)], dstb, sem)
            pltpu.async_copy(src_h.at[pl.ds(brow, SROWS)], srcb, sem)
            pltpu.make_async_copy(dst_h.at[pl.ds(brow, SROWS)], dstb, sem).wait()
            pltpu.make_async_copy(src_h.at[pl.ds(brow, SROWS)], srcb, sem).wait()
            pltpu.sync_copy(val_h.at[pl.ds(brow, SROWS)], valb)
            for q in range(NGB - 1):
                pltpu.async_copy(src_tab.at[c].at[srcb.at[q]], gbufs[q],
                                 gsems[q])

            def ring(rj, pcarry):
                for par in range(NGB):
                    j = rj * NGB + par
                    gbuf, gsem = gbufs[par], gsems[par]
                    sbuf, ssem = sbufs[par % NSB], ssems[par % NSB]
                    f = j + NGB - 1
                    q = (par + NGB - 1) % NGB

                    if par == 0:
                        pltpu.async_copy(
                            src_tab.at[c].at[srcb.at[f]], gbufs[q], gsems[q])
                    else:
                        @pl.when(f < SROWS)
                        def _():
                            pltpu.async_copy(
                                src_tab.at[c].at[srcb.at[f]], gbufs[q],
                                gsems[q])

                    pltpu.make_async_copy(
                        src_tab.at[c].at[srcb.at[j]], gbuf, gsem).wait()

                    if par < NSB:
                        @pl.when(j >= NSB)
                        def _():
                            pltpu.make_async_copy(
                                sbuf, acc_sh.at[dstb.at[j - NSB]], ssem).wait()
                    else:
                        pltpu.make_async_copy(
                            sbuf, acc_sh.at[dstb.at[j - NSB]], ssem).wait()

                    @plsc.parallel_loop(0, CHUNK, 16)
                    def _(e):
                        vv = valb[j, pl.ds(e, 16)]
                        for l in range(16):
                            w = jnp.full((16,), vv[l], jnp.float32)
                            sbuf[e + l, pl.ds(0, 16)] = (
                                gbuf[e + l, pl.ds(0, 16)] * w)
                            sbuf[e + l, pl.ds(16, 16)] = (
                                gbuf[e + l, pl.ds(16, 16)] * w)

                    pltpu.async_copy(sbuf, acc_sh.at[dstb.at[j]], ssem,
                                     add=True)
                return pcarry
            lax.fori_loop(0, SROWS // NGB, ring, 0)
            for j in range(SROWS - NSB, SROWS):
                pltpu.make_async_copy(
                    sbufs[j % NSB], acc_sh.at[dstb.at[j]],
                    ssems[j % NSB]).wait()
            return carry
        lax.fori_loop(0, n_super, echunk, 0)
        plsc.subcore_barrier()

        def pcopy(j, carry):
            pltpu.sync_copy(acc_sh.at[pl.ds(rbase + j * CP, CP)],
                            s_o.at[c, pl.ds(rbase + j * CP, CP)])
            return carry
        lax.fori_loop(0, ROWS_PT // CP, pcopy, 0)
        plsc.subcore_barrier()

    samp_accum(0)
    for k in range(1, HOPS + 1):
        hop(k == 1, acc_sh)
        samp_accum(k)


def _loss_body(u_ref, p_ref, n_ref, o_ref):
    u = jnp.sum(u_ref[...], axis=1)
    p = jnp.sum(p_ref[...], axis=1)
    n = jnp.sum(n_ref[...], axis=1)
    scale = 1.0 / float(HOPS + 1)
    s2 = scale * scale
    ps = jnp.sum(jnp.sum(u * p, axis=-1), axis=0) * s2
    ns = jnp.sum(jnp.sum(u * n, axis=-1), axis=0) * s2
    mf = jnp.mean(jnp.log(1.0 + jnp.exp(ns - ps)))
    sq = (jnp.sum(u * u) + jnp.sum(p * p) + jnp.sum(n * n)) * s2
    loss = mf + DECAY * (sq * 0.5) / float(B)
    o_ref[...] = jnp.full((8, 128), loss, jnp.float32)


def kernel(genes, pos_items, neg_items, gene_embed, drug_embed, gene_t,
           drug_t, adj_indices, adj_values):
    E = adj_values.shape[0]
    blk = NS * SUPER
    epad = ((E + blk - 1) // blk) * blk
    dst = jnp.pad(adj_indices[0], (0, epad - E)).reshape(-1, CHUNK)
    src = jnp.pad(adj_indices[1], (0, epad - E)).reshape(-1, CHUNK)
    val = jnp.pad(adj_values, (0, epad - E)).reshape(-1, CHUNK)

    all_embed = jnp.concatenate([gene_embed, drug_embed], axis=0)
    all_embed = jnp.pad(all_embed, ((0, NPAD - N), (0, 0)))
    tab = jnp.stack([all_embed[:, :HALF], all_embed[:, HALF:]], axis=0)
    t_all = jnp.concatenate([gene_t[:, 0], drug_t[:, 0]], axis=0)
    neg0 = neg_items[:, 0]

    mesh = plsc.VectorSubcoreMesh(core_axis_name="c", subcore_axis_name="s",
                                  num_cores=NC, num_subcores=NS)
    f32 = jnp.float32
    sc = pl.kernel(
        functools.partial(_sc_body, epad),
        out_type=(
            jax.ShapeDtypeStruct((NC, HOPS + 1, B, HALF), f32),
            jax.ShapeDtypeStruct((NC, HOPS + 1, B, HALF), f32),
            jax.ShapeDtypeStruct((NC, HOPS + 1, B, HALF), f32),
            jax.ShapeDtypeStruct((NC, NPAD, HALF), f32),
        ),
        mesh=mesh,
        compiler_params=pltpu.CompilerParams(use_tc_tiling_on_sc=False),
        scratch_types=[
            pltpu.VMEM((SROWS, CHUNK), jnp.int32),
            pltpu.VMEM((SROWS, CHUNK), jnp.int32),
            pltpu.VMEM((SROWS, CHUNK), f32),
            pltpu.VMEM((CHUNK, HALF), f32),
            pltpu.VMEM((CHUNK, HALF), f32),
            pltpu.VMEM((CHUNK, HALF), f32),
            pltpu.VMEM((CHUNK, HALF), f32),
            pltpu.VMEM((CHUNK, HALF), f32),
            pltpu.VMEM((CHUNK, HALF), f32),
            pltpu.VMEM((SAMP,), jnp.int32),
            pltpu.VMEM((SAMP,), jnp.int32),
            pltpu.VMEM((SAMP,), jnp.int32),
            pltpu.VMEM((SAMP,), f32),
            pltpu.VMEM((SAMP,), f32),
            pltpu.VMEM((SAMP,), f32),
            pltpu.VMEM((SAMP,), f32),
            pltpu.VMEM_SHARED((NPAD, HALF), f32),
        ] + [pltpu.SemaphoreType.DMA] * 7,
    )
    u8, p8, n8, _ = sc(tab, dst, src, val, t_all, genes, pos_items, neg0)

    loss = pl.pallas_call(
        _loss_body,
        out_shape=jax.ShapeDtypeStruct((8, 128), f32),
    )(u8, p8, n8)
    return loss[0, 0]


# speedup vs baseline: 1.4005x; 1.4005x over previous
import functools
import math

import jax
import jax.numpy as jnp
from jax import lax
from jax.experimental import pallas as pl
from jax.experimental.pallas import tpu as pltpu
from jax.experimental.pallas import tpu_sc as plsc

N_GENES = 40000
N_DRUGS = 10000
N = N_GENES + N_DRUGS
D = 64
HALF = 32
HOPS = 3
B = 4096
DECAY = 1e-4

NC = 2
NS = 16
CHUNK = 128
SUPER = 1024
SROWS = SUPER // CHUNK
NBUF = 4
SAMP = B // NS
NPAD = 50176
ROWS_PT = NPAD // NS
CP = 112


def _sc_body(epad, tab_h, dst_h, src_h, val_h, tall_h, g_h, p_h, n_h,
             u_o, po_o, ne_o, s_o,
             dstb, srcb, valb, rows, rows1, rows2, rows3, gidx, pidx, nidx,
             gt, pt, nt, wbuf, acc_sh, sem, semg0, semg1, semg2, semg3,
             sems0, sems1, sems2, sems3):
    c = lax.axis_index("c")
    s = lax.axis_index("s")
    sbase = s * SAMP
    ept = epad // NS
    n_super = ept // SUPER
    erow0 = s * (ept // CHUNK)

    kinds = ((gidx, gt, u_o, g_h, 0), (pidx, pt, po_o, p_h, N_GENES),
             (nidx, nt, ne_o, n_h, N_GENES))

    for idx, tv, _, src_h_k, off in kinds:
        pltpu.sync_copy(src_h_k.at[pl.ds(sbase, SAMP)], idx)
        if off:
            @plsc.parallel_loop(0, SAMP, 16)
            def _(i):
                idx[pl.ds(i, 16)] = idx[pl.ds(i, 16)] + off
        for j in range(SAMP // CHUNK):
            pltpu.async_copy(
                tall_h.at[idx.at[pl.ds(j * CHUNK, CHUNK)]],
                tv.at[pl.ds(j * CHUNK, CHUNK)], sem).wait()

    def samp_accum(k):
        fk = 1.0 / float(math.factorial(k))
        src_tab = tab_h if k == 0 else s_o
        for idx, tv, out, _, _ in kinds:
            @plsc.parallel_loop(0, SAMP, 16)
            def _(i):
                t = tv[pl.ds(i, 16)]
                tk = jnp.full((16,), fk, jnp.float32)
                for _ in range(k):
                    tk = tk * t
                wbuf[pl.ds(i, 16)] = jnp.exp(-t) * tk

            def sjloop(j, scarry):
                pltpu.async_copy(
                    src_tab.at[c].at[idx.at[pl.ds(j * CHUNK, CHUNK)]],
                    rows, sem).wait()

                @plsc.parallel_loop(0, CHUNK, 16)
                def _(i):
                    wv = wbuf[pl.ds(j * CHUNK + i, 16)]
                    for l in range(16):
                        w = jnp.full((16,), wv[l], jnp.float32)
                        rows[i + l, pl.ds(0, 16)] = rows[i + l, pl.ds(0, 16)] * w
                        rows[i + l, pl.ds(16, 16)] = (
                            rows[i + l, pl.ds(16, 16)] * w)

                pltpu.sync_copy(
                    rows, out.at[c, k, pl.ds(sbase + j * CHUNK, CHUNK)])
                return scarry
            lax.fori_loop(0, SAMP // CHUNK, sjloop, 0)

    def hop(first, acc_sh):
        src_tab = tab_h if first else s_o

        @plsc.parallel_loop(0, CHUNK, 1, unroll=8)
        def _(i):
            z = jnp.zeros((16,), jnp.float32)
            rows[i, pl.ds(0, 16)] = z
            rows[i, pl.ds(16, 16)] = z

        rbase = s * ROWS_PT

        def zcopy(j, carry):
            pltpu.sync_copy(rows.at[pl.ds(0, CP)],
                            acc_sh.at[pl.ds(rbase + j * CP, CP)])
            return carry
        lax.fori_loop(0, ROWS_PT // CP, zcopy, 0)
        plsc.subcore_barrier()

        bufs = (rows, rows1, rows2, rows3)
        sems = (semg0, semg1, semg2, semg3)
        ssems = (sems0, sems1, sems2, sems3)

        def echunk(sc_i, carry):
            brow = erow0 + sc_i * SROWS
            pltpu.async_copy(dst_h.at[pl.ds(brow, SROWS)], dstb, sem)
            pltpu.async_copy(src_h.at[pl.ds(brow, SROWS)], srcb, sem)
            pltpu.make_async_copy(dst_h.at[pl.ds(brow, SROWS)], dstb, sem).wait()
            pltpu.make_async_copy(src_h.at[pl.ds(brow, SROWS)], srcb, sem).wait()
            pltpu.sync_copy(val_h.at[pl.ds(brow, SROWS)], valb)
            for q in range(NBUF - 1):
                pltpu.async_copy(src_tab.at[c].at[srcb.at[q]], bufs[q], sems[q])

            def ring(rj, pcarry):
                for par in range(NBUF):
                    j = rj * NBUF + par
                    buf, bsem, bss = bufs[par], sems[par], ssems[par]
                    f = j + NBUF - 1
                    q = (par + NBUF - 1) % NBUF
                    fbuf, fsem, fss = bufs[q], sems[q], ssems[q]

                    if par == 0:
                        @pl.when(j >= 1)
                        def _():
                            pltpu.make_async_copy(
                                fbuf, acc_sh.at[dstb.at[j - 1]], fss).wait()
                        pltpu.async_copy(
                            src_tab.at[c].at[srcb.at[f]], fbuf, fsem)
                    else:
                        @pl.when(f < SROWS)
                        def _():
                            pltpu.make_async_copy(
                                fbuf, acc_sh.at[dstb.at[j - 1]], fss).wait()
                            pltpu.async_copy(
                                src_tab.at[c].at[srcb.at[f]], fbuf, fsem)

                    pltpu.make_async_copy(
                        src_tab.at[c].at[srcb.at[j]], buf, bsem).wait()

                    @plsc.parallel_loop(0, CHUNK, 16)
                    def _(e):
                        vv = valb[j, pl.ds(e, 16)]
                        for l in range(16):
                            w = jnp.full((16,), vv[l], jnp.float32)
                            buf[e + l, pl.ds(0, 16)] = (
                                buf[e + l, pl.ds(0, 16)] * w)
                            buf[e + l, pl.ds(16, 16)] = (
                                buf[e + l, pl.ds(16, 16)] * w)

                    pltpu.async_copy(buf, acc_sh.at[dstb.at[j]], bss, add=True)
                return pcarry
            lax.fori_loop(0, SROWS // NBUF, ring, 0)
            for j in range(SROWS - NBUF, SROWS):
                pltpu.make_async_copy(
                    bufs[j % NBUF], acc_sh.at[dstb.at[j]],
                    ssems[j % NBUF]).wait()
            return carry
        lax.fori_loop(0, n_super, echunk, 0)
        plsc.subcore_barrier()

        def pcopy(j, carry):
            pltpu.sync_copy(acc_sh.at[pl.ds(rbase + j * CP, CP)],
                            s_o.at[c, pl.ds(rbase + j * CP, CP)])
            return carry
        lax.fori_loop(0, ROWS_PT // CP, pcopy, 0)
        plsc.subcore_barrier()

    samp_accum(0)
    for k in range(1, HOPS + 1):
        hop(k == 1, acc_sh)
        samp_accum(k)


def _loss_body(u_ref, p_ref, n_ref, o_ref):
    u = jnp.sum(u_ref[...], axis=1)
    p = jnp.sum(p_ref[...], axis=1)
    n = jnp.sum(n_ref[...], axis=1)
    scale = 1.0 / float(HOPS + 1)
    s2 = scale * scale
    ps = jnp.sum(jnp.sum(u * p, axis=-1), axis=0) * s2
    ns = jnp.sum(jnp.sum(u * n, axis=-1), axis=0) * s2
    mf = jnp.mean(jnp.log(1.0 + jnp.exp(ns - ps)))
    sq = (jnp.sum(u * u) + jnp.sum(p * p) + jnp.sum(n * n)) * s2
    loss = mf + DECAY * (sq * 0.5) / float(B)
    o_ref[...] = jnp.full((8, 128), loss, jnp.float32)


def kernel(genes, pos_items, neg_items, gene_embed, drug_embed, gene_t,
           drug_t, adj_indices, adj_values):
    E = adj_values.shape[0]
    blk = NS * SUPER
    epad = ((E + blk - 1) // blk) * blk
    dst = jnp.pad(adj_indices[0], (0, epad - E)).reshape(-1, CHUNK)
    src = jnp.pad(adj_indices[1], (0, epad - E)).reshape(-1, CHUNK)
    val = jnp.pad(adj_values, (0, epad - E)).reshape(-1, CHUNK)

    all_embed = jnp.concatenate([gene_embed, drug_embed], axis=0)
    all_embed = jnp.pad(all_embed, ((0, NPAD - N), (0, 0)))
    tab = jnp.stack([all_embed[:, :HALF], all_embed[:, HALF:]], axis=0)
    t_all = jnp.concatenate([gene_t[:, 0], drug_t[:, 0]], axis=0)
    neg0 = neg_items[:, 0]

    mesh = plsc.VectorSubcoreMesh(core_axis_name="c", subcore_axis_name="s",
                                  num_cores=NC, num_subcores=NS)
    f32 = jnp.float32
    sc = pl.kernel(
        functools.partial(_sc_body, epad),
        out_type=(
            jax.ShapeDtypeStruct((NC, HOPS + 1, B, HALF), f32),
            jax.ShapeDtypeStruct((NC, HOPS + 1, B, HALF), f32),
            jax.ShapeDtypeStruct((NC, HOPS + 1, B, HALF), f32),
            jax.ShapeDtypeStruct((NC, NPAD, HALF), f32),
        ),
        mesh=mesh,
        compiler_params=pltpu.CompilerParams(use_tc_tiling_on_sc=False),
        scratch_types=[
            pltpu.VMEM((SROWS, CHUNK), jnp.int32),
            pltpu.VMEM((SROWS, CHUNK), jnp.int32),
            pltpu.VMEM((SROWS, CHUNK), f32),
            pltpu.VMEM((CHUNK, HALF), f32),
            pltpu.VMEM((CHUNK, HALF), f32),
            pltpu.VMEM((CHUNK, HALF), f32),
            pltpu.VMEM((CHUNK, HALF), f32),
            pltpu.VMEM((SAMP,), jnp.int32),
            pltpu.VMEM((SAMP,), jnp.int32),
            pltpu.VMEM((SAMP,), jnp.int32),
            pltpu.VMEM((SAMP,), f32),
            pltpu.VMEM((SAMP,), f32),
            pltpu.VMEM((SAMP,), f32),
            pltpu.VMEM((SAMP,), f32),
            pltpu.VMEM_SHARED((NPAD, HALF), f32),
        ] + [pltpu.SemaphoreType.DMA] * 9,
    )
    u8, p8, n8, _ = sc(tab, dst, src, val, t_all, genes, pos_items, neg0)

    loss = pl.pallas_call(
        _loss_body,
        out_shape=jax.ShapeDtypeStruct((8, 128), f32),
    )(u8, p8, n8)
    return loss[0, 0]


# speedup vs baseline: 1.4415x; 1.0293x over previous
import functools
import math

import jax
import jax.numpy as jnp
from jax import lax
from jax.experimental import pallas as pl
from jax.experimental.pallas import tpu as pltpu
from jax.experimental.pallas import tpu_sc as plsc

N_GENES = 40000
N_DRUGS = 10000
N = N_GENES + N_DRUGS
D = 64
HALF = 32
HOPS = 3
B = 4096
DECAY = 1e-4

NC = 2
NS = 16
CHUNK = 128
SUPER = 1024
SROWS = SUPER // CHUNK
NBUF = 4
SAMP = B // NS
NPAD = 50176
ROWS_PT = NPAD // NS
CP = 112


def _sc_body(epad, tab_h, dst_h, src_h, val_h, tall_h, g_h, p_h, n_h,
             u_o, po_o, ne_o, s_o,
             dstb, srcb, valb, rows, rows1, rows2, rows3, gidx, pidx, nidx,
             gt, pt, nt, wbuf, acc_sh, sem, semg0, semg1, semg2, semg3,
             sems0, sems1, sems2, sems3):
    c = lax.axis_index("c")
    s = lax.axis_index("s")
    sbase = s * SAMP
    ept = epad // NS
    n_super = ept // SUPER
    erow0 = s * (ept // CHUNK)

    kinds = ((gidx, gt, u_o, g_h, 0), (pidx, pt, po_o, p_h, N_GENES),
             (nidx, nt, ne_o, n_h, N_GENES))

    for idx, tv, _, src_h_k, off in kinds:
        pltpu.sync_copy(src_h_k.at[pl.ds(sbase, SAMP)], idx)
        if off:
            @plsc.parallel_loop(0, SAMP, 16)
            def _(i):
                idx[pl.ds(i, 16)] = idx[pl.ds(i, 16)] + off
        for j in range(SAMP // CHUNK):
            pltpu.async_copy(
                tall_h.at[idx.at[pl.ds(j * CHUNK, CHUNK)]],
                tv.at[pl.ds(j * CHUNK, CHUNK)], sem).wait()

    def samp_accum(k):
        fk = 1.0 / float(math.factorial(k))
        src_tab = tab_h if k == 0 else s_o
        for idx, tv, out, _, _ in kinds:
            @plsc.parallel_loop(0, SAMP, 16)
            def _(i):
                t = tv[pl.ds(i, 16)]
                tk = jnp.full((16,), fk, jnp.float32)
                for _ in range(k):
                    tk = tk * t
                wbuf[pl.ds(i, 16)] = jnp.exp(-t) * tk

            for j in range(SAMP // CHUNK):
                pltpu.async_copy(
                    src_tab.at[c].at[idx.at[pl.ds(j * CHUNK, CHUNK)]],
                    rows1, sem)
                pltpu.make_async_copy(
                    src_tab.at[c].at[idx.at[pl.ds(j * CHUNK, CHUNK)]],
                    rows1, sem).wait()

                @plsc.parallel_loop(0, CHUNK, 16)
                def _(i):
                    wv = wbuf[pl.ds(j * CHUNK + i, 16)]
                    for l in range(16):
                        w = jnp.full((16,), wv[l], jnp.float32)
                        rows1[i + l, pl.ds(0, 16)] = (
                            rows1[i + l, pl.ds(0, 16)] * w)
                        rows1[i + l, pl.ds(16, 16)] = (
                            rows1[i + l, pl.ds(16, 16)] * w)

                pltpu.sync_copy(
                    rows1, out.at[c, k, pl.ds(sbase + j * CHUNK, CHUNK)])

    def hop(first, acc_sh):
        src_tab = tab_h if first else s_o

        @plsc.parallel_loop(0, CHUNK, 1, unroll=8)
        def _(i):
            z = jnp.zeros((16,), jnp.float32)
            rows[i, pl.ds(0, 16)] = z
            rows[i, pl.ds(16, 16)] = z

        rbase = s * ROWS_PT

        def zcopy(j, carry):
            pltpu.async_copy(rows.at[pl.ds(0, CP)],
                             acc_sh.at[pl.ds(rbase + j * CP, CP)], sem)
            return carry
        lax.fori_loop(0, ROWS_PT // CP, zcopy, 0)

        def zdrain(j, carry):
            pltpu.make_async_copy(rows.at[pl.ds(0, CP)],
                                  acc_sh.at[pl.ds(rbase + j * CP, CP)],
                                  sem).wait()
            return carry
        lax.fori_loop(0, ROWS_PT // CP, zdrain, 0)
        plsc.subcore_barrier()

        bufs = (rows, rows1, rows2, rows3)
        sems = (semg0, semg1, semg2, semg3)
        ssems = (sems0, sems1, sems2, sems3)

        def echunk(sc_i, carry):
            brow = erow0 + sc_i * SROWS
            pltpu.async_copy(dst_h.at[pl.ds(brow, SROWS)], dstb, sem)
            pltpu.async_copy(src_h.at[pl.ds(brow, SROWS)], srcb, sem)
            pltpu.make_async_copy(dst_h.at[pl.ds(brow, SROWS)], dstb, sem).wait()
            pltpu.make_async_copy(src_h.at[pl.ds(brow, SROWS)], srcb, sem).wait()
            pltpu.sync_copy(val_h.at[pl.ds(brow, SROWS)], valb)
            for q in range(NBUF - 1):
                pltpu.async_copy(src_tab.at[c].at[srcb.at[q]], bufs[q], sems[q])

            def ring(rj, pcarry):
                for par in range(NBUF):
                    j = rj * NBUF + par
                    buf, bsem, bss = bufs[par], sems[par], ssems[par]
                    f = j + NBUF - 1
                    q = (par + NBUF - 1) % NBUF
                    fbuf, fsem, fss = bufs[q], sems[q], ssems[q]

                    if par == 0:
                        @pl.when(j >= 1)
                        def _():
                            pltpu.make_async_copy(
                                fbuf, acc_sh.at[dstb.at[j - 1]], fss).wait()
                        pltpu.async_copy(
                            src_tab.at[c].at[srcb.at[f]], fbuf, fsem)
                    else:
                        @pl.when(f < SROWS)
                        def _():
                            pltpu.make_async_copy(
                                fbuf, acc_sh.at[dstb.at[j - 1]], fss).wait()
                            pltpu.async_copy(
                                src_tab.at[c].at[srcb.at[f]], fbuf, fsem)

                    pltpu.make_async_copy(
                        src_tab.at[c].at[srcb.at[j]], buf, bsem).wait()

                    @plsc.parallel_loop(0, CHUNK, 16)
                    def _(e):
                        vv = valb[j, pl.ds(e, 16)]
                        for l in range(16):
                            w = jnp.full((16,), vv[l], jnp.float32)
                            buf[e + l, pl.ds(0, 16)] = (
                                buf[e + l, pl.ds(0, 16)] * w)
                            buf[e + l, pl.ds(16, 16)] = (
                                buf[e + l, pl.ds(16, 16)] * w)

                    pltpu.async_copy(buf, acc_sh.at[dstb.at[j]], bss, add=True)
                return pcarry
            lax.fori_loop(0, SROWS // NBUF, ring, 0)
            for j in range(SROWS - NBUF, SROWS):
                pltpu.make_async_copy(
                    bufs[j % NBUF], acc_sh.at[dstb.at[j]],
                    ssems[j % NBUF]).wait()
            return carry
        lax.fori_loop(0, n_super, echunk, 0)
        plsc.subcore_barrier()

        def pcopy(j, carry):
            pltpu.async_copy(acc_sh.at[pl.ds(rbase + j * CP, CP)],
                             s_o.at[c, pl.ds(rbase + j * CP, CP)], sem)
            return carry
        lax.fori_loop(0, ROWS_PT // CP, pcopy, 0)

        def pdrain(j, carry):
            pltpu.make_async_copy(acc_sh.at[pl.ds(rbase + j * CP, CP)],
                                  s_o.at[c, pl.ds(rbase + j * CP, CP)],
                                  sem).wait()
            return carry
        lax.fori_loop(0, ROWS_PT // CP, pdrain, 0)
        plsc.subcore_barrier()

    samp_accum(0)
    for k in range(1, HOPS + 1):
        hop(k == 1, acc_sh)
        samp_accum(k)


def _loss_body(u_ref, p_ref, n_ref, o_ref):
    u = jnp.sum(u_ref[...], axis=1)
    p = jnp.sum(p_ref[...], axis=1)
    n = jnp.sum(n_ref[...], axis=1)
    scale = 1.0 / float(HOPS + 1)
    s2 = scale * scale
    ps = jnp.sum(jnp.sum(u * p, axis=-1), axis=0) * s2
    ns = jnp.sum(jnp.sum(u * n, axis=-1), axis=0) * s2
    mf = jnp.mean(jnp.log(1.0 + jnp.exp(ns - ps)))
    sq = (jnp.sum(u * u) + jnp.sum(p * p) + jnp.sum(n * n)) * s2
    loss = mf + DECAY * (sq * 0.5) / float(B)
    o_ref[...] = jnp.full((8, 128), loss, jnp.float32)


def kernel(genes, pos_items, neg_items, gene_embed, drug_embed, gene_t,
           drug_t, adj_indices, adj_values):
    E = adj_values.shape[0]
    blk = NS * SUPER
    epad = ((E + blk - 1) // blk) * blk
    dst = jnp.pad(adj_indices[0], (0, epad - E)).reshape(-1, CHUNK)
    src = jnp.pad(adj_indices[1], (0, epad - E)).reshape(-1, CHUNK)
    val = jnp.pad(adj_values, (0, epad - E)).reshape(-1, CHUNK)

    all_embed = jnp.concatenate([gene_embed, drug_embed], axis=0)
    all_embed = jnp.pad(all_embed, ((0, NPAD - N), (0, 0)))
    tab = jnp.stack([all_embed[:, :HALF], all_embed[:, HALF:]], axis=0)
    t_all = jnp.concatenate([gene_t[:, 0], drug_t[:, 0]], axis=0)
    neg0 = neg_items[:, 0]

    mesh = plsc.VectorSubcoreMesh(core_axis_name="c", subcore_axis_name="s",
                                  num_cores=NC, num_subcores=NS)
    f32 = jnp.float32
    sc = pl.kernel(
        functools.partial(_sc_body, epad),
        out_type=(
            jax.ShapeDtypeStruct((NC, HOPS + 1, B, HALF), f32),
            jax.ShapeDtypeStruct((NC, HOPS + 1, B, HALF), f32),
            jax.ShapeDtypeStruct((NC, HOPS + 1, B, HALF), f32),
            jax.ShapeDtypeStruct((NC, NPAD, HALF), f32),
        ),
        mesh=mesh,
        compiler_params=pltpu.CompilerParams(use_tc_tiling_on_sc=False),
        scratch_types=[
            pltpu.VMEM((SROWS, CHUNK), jnp.int32),
            pltpu.VMEM((SROWS, CHUNK), jnp.int32),
            pltpu.VMEM((SROWS, CHUNK), f32),
            pltpu.VMEM((CHUNK, HALF), f32),
            pltpu.VMEM((CHUNK, HALF), f32),
            pltpu.VMEM((CHUNK, HALF), f32),
            pltpu.VMEM((CHUNK, HALF), f32),
            pltpu.VMEM((SAMP,), jnp.int32),
            pltpu.VMEM((SAMP,), jnp.int32),
            pltpu.VMEM((SAMP,), jnp.int32),
            pltpu.VMEM((SAMP,), f32),
            pltpu.VMEM((SAMP,), f32),
            pltpu.VMEM((SAMP,), f32),
            pltpu.VMEM((SAMP,), f32),
            pltpu.VMEM_SHARED((NPAD, HALF), f32),
        ] + [pltpu.SemaphoreType.DMA] * 9,
    )
    u8, p8, n8, _ = sc(tab, dst, src, val, t_all, genes, pos_items, neg0)

    loss = pl.pallas_call(
        _loss_body,
        out_shape=jax.ShapeDtypeStruct((8, 128), f32),
    )(u8, p8, n8)
    return loss[0, 0]


# speedup vs baseline: 1.4849x; 1.0301x over previous
import functools
import math

import jax
import jax.numpy as jnp
from jax import lax
from jax.experimental import pallas as pl
from jax.experimental.pallas import tpu as pltpu
from jax.experimental.pallas import tpu_sc as plsc

N_GENES = 40000
N_DRUGS = 10000
N = N_GENES + N_DRUGS
D = 64
HALF = 32
HOPS = 3
B = 4096
DECAY = 1e-4

NC = 2
NS = 16
CHUNK = 128
SUPER = 1024
SROWS = SUPER // CHUNK
NBUF = 4
SAMP = B // NS
NPAD = 50176
ROWS_PT = NPAD // NS
CP = 112


def _sc_body(epad, tab_h, dst_h, src_h, val_h, tall_h, g_h, p_h, n_h,
             u_o, po_o, ne_o, s_o,
             dstb, srcb, valb, rows, rows1, rows2, rows3, gidx, pidx, nidx,
             gt, pt, nt, wbuf, acc_sh, sem, semg0, semg1, semg2, semg3,
             sems0, sems1, sems2, sems3):
    c = lax.axis_index("c")
    s = lax.axis_index("s")
    sbase = s * SAMP
    ept = epad // NS
    n_super = ept // SUPER
    erow0 = s * (ept // CHUNK)

    kinds = ((gidx, gt, u_o, g_h, 0), (pidx, pt, po_o, p_h, N_GENES),
             (nidx, nt, ne_o, n_h, N_GENES))

    for idx, tv, _, src_h_k, off in kinds:
        pltpu.sync_copy(src_h_k.at[pl.ds(sbase, SAMP)], idx)
        if off:
            @plsc.parallel_loop(0, SAMP, 16)
            def _(i):
                idx[pl.ds(i, 16)] = idx[pl.ds(i, 16)] + off
        for j in range(SAMP // CHUNK):
            pltpu.async_copy(
                tall_h.at[idx.at[pl.ds(j * CHUNK, CHUNK)]],
                tv.at[pl.ds(j * CHUNK, CHUNK)], sem).wait()

    def samp_accum(k):
        fk = 1.0 / float(math.factorial(k))
        src_tab = tab_h if k == 0 else s_o
        for idx, tv, out, _, _ in kinds:
            @plsc.parallel_loop(0, SAMP, 16)
            def _(i):
                t = tv[pl.ds(i, 16)]
                tk = jnp.full((16,), fk, jnp.float32)
                for _ in range(k):
                    tk = tk * t
                wbuf[pl.ds(i, 16)] = jnp.exp(-t) * tk

            for j in range(SAMP // CHUNK):
                pltpu.async_copy(
                    src_tab.at[c].at[idx.at[pl.ds(j * CHUNK, CHUNK)]],
                    rows1, sem)
                pltpu.make_async_copy(
                    src_tab.at[c].at[idx.at[pl.ds(j * CHUNK, CHUNK)]],
                    rows1, sem).wait()

                @plsc.parallel_loop(0, CHUNK, 16)
                def _(i):
                    wv = wbuf[pl.ds(j * CHUNK + i, 16)]
                    for l in range(16):
                        w = jnp.full((16,), wv[l], jnp.float32)
                        rows1[i + l, pl.ds(0, 16)] = (
                            rows1[i + l, pl.ds(0, 16)] * w)
                        rows1[i + l, pl.ds(16, 16)] = (
                            rows1[i + l, pl.ds(16, 16)] * w)

                pltpu.sync_copy(
                    rows1, out.at[c, k, pl.ds(sbase + j * CHUNK, CHUNK)])

    def hop(first, acc_sh):
        src_tab = tab_h if first else s_o

        @plsc.parallel_loop(0, CHUNK, 1, unroll=8)
        def _(i):
            z = jnp.zeros((16,), jnp.float32)
            rows[i, pl.ds(0, 16)] = z
            rows[i, pl.ds(16, 16)] = z

        rbase = s * ROWS_PT

        def zcopy(j, carry):
            pltpu.async_copy(rows.at[pl.ds(0, CP)],
                             acc_sh.at[pl.ds(rbase + j * CP, CP)], sem)
            return carry
        lax.fori_loop(0, ROWS_PT // CP, zcopy, 0)

        def zdrain(j, carry):
            pltpu.make_async_copy(rows.at[pl.ds(0, CP)],
                                  acc_sh.at[pl.ds(rbase + j * CP, CP)],
                                  sem).wait()
            return carry
        lax.fori_loop(0, ROWS_PT // CP, zdrain, 0)
        plsc.subcore_barrier()

        bufs = (rows, rows1, rows2, rows3)
        sems = (semg0, semg1, semg2, semg3)
        ssems = (sems0, sems1, sems2, sems3)

        def echunk(sc_i, carry):
            brow = erow0 + sc_i * SROWS
            pltpu.async_copy(dst_h.at[pl.ds(brow, SROWS)], dstb, sem)
            pltpu.async_copy(src_h.at[pl.ds(brow, SROWS)], srcb, sem)
            pltpu.make_async_copy(dst_h.at[pl.ds(brow, SROWS)], dstb, sem).wait()
            pltpu.make_async_copy(src_h.at[pl.ds(brow, SROWS)], srcb, sem).wait()
            pltpu.sync_copy(val_h.at[pl.ds(brow, SROWS)], valb)
            for q in range(NBUF - 1):
                pltpu.async_copy(src_tab.at[c].at[srcb.at[q]], bufs[q], sems[q])

            def ring(rj, pcarry):
                for par in range(NBUF):
                    j = rj * NBUF + par
                    buf, bsem, bss = bufs[par], sems[par], ssems[par]
                    f = j + NBUF - 1
                    q = (par + NBUF - 1) % NBUF
                    fbuf, fsem, fss = bufs[q], sems[q], ssems[q]

                    pltpu.make_async_copy(
                        src_tab.at[c].at[srcb.at[j]], buf, bsem).wait()

                    @plsc.parallel_loop(0, CHUNK, 16)
                    def _(e):
                        vv = valb[j, pl.ds(e, 16)]
                        for l in range(16):
                            w = jnp.full((16,), vv[l], jnp.float32)
                            buf[e + l, pl.ds(0, 16)] = (
                                buf[e + l, pl.ds(0, 16)] * w)
                            buf[e + l, pl.ds(16, 16)] = (
                                buf[e + l, pl.ds(16, 16)] * w)

                    if par == 0:
                        @pl.when(j >= 1)
                        def _():
                            pltpu.make_async_copy(
                                fbuf, acc_sh.at[dstb.at[j - 1]], fss).wait()
                        pltpu.async_copy(
                            src_tab.at[c].at[srcb.at[f]], fbuf, fsem)
                    else:
                        @pl.when(f < SROWS)
                        def _():
                            pltpu.make_async_copy(
                                fbuf, acc_sh.at[dstb.at[j - 1]], fss).wait()
                            pltpu.async_copy(
                                src_tab.at[c].at[srcb.at[f]], fbuf, fsem)

                    pltpu.async_copy(buf, acc_sh.at[dstb.at[j]], bss, add=True)
                return pcarry
            lax.fori_loop(0, SROWS // NBUF, ring, 0)
            for j in range(SROWS - NBUF, SROWS):
                pltpu.make_async_copy(
                    bufs[j % NBUF], acc_sh.at[dstb.at[j]],
                    ssems[j % NBUF]).wait()
            return carry
        lax.fori_loop(0, n_super, echunk, 0)
        plsc.subcore_barrier()

        def pcopy(j, carry):
            pltpu.async_copy(acc_sh.at[pl.ds(rbase + j * CP, CP)],
                             s_o.at[c, pl.ds(rbase + j * CP, CP)], sem)
            return carry
        lax.fori_loop(0, ROWS_PT // CP, pcopy, 0)

        def pdrain(j, carry):
            pltpu.make_async_copy(acc_sh.at[pl.ds(rbase + j * CP, CP)],
                                  s_o.at[c, pl.ds(rbase + j * CP, CP)],
                                  sem).wait()
            return carry
        lax.fori_loop(0, ROWS_PT // CP, pdrain, 0)
        plsc.subcore_barrier()

    samp_accum(0)
    for k in range(1, HOPS + 1):
        hop(k == 1, acc_sh)
        samp_accum(k)


def _loss_body(u_ref, p_ref, n_ref, o_ref):
    u = jnp.sum(u_ref[...], axis=1)
    p = jnp.sum(p_ref[...], axis=1)
    n = jnp.sum(n_ref[...], axis=1)
    scale = 1.0 / float(HOPS + 1)
    s2 = scale * scale
    ps = jnp.sum(jnp.sum(u * p, axis=-1), axis=0) * s2
    ns = jnp.sum(jnp.sum(u * n, axis=-1), axis=0) * s2
    mf = jnp.mean(jnp.log(1.0 + jnp.exp(ns - ps)))
    sq = (jnp.sum(u * u) + jnp.sum(p * p) + jnp.sum(n * n)) * s2
    loss = mf + DECAY * (sq * 0.5) / float(B)
    o_ref[...] = jnp.full((8, 128), loss, jnp.float32)


def kernel(genes, pos_items, neg_items, gene_embed, drug_embed, gene_t,
           drug_t, adj_indices, adj_values):
    E = adj_values.shape[0]
    blk = NS * SUPER
    epad = ((E + blk - 1) // blk) * blk
    dst = jnp.pad(adj_indices[0], (0, epad - E)).reshape(-1, CHUNK)
    src = jnp.pad(adj_indices[1], (0, epad - E)).reshape(-1, CHUNK)
    val = jnp.pad(adj_values, (0, epad - E)).reshape(-1, CHUNK)

    all_embed = jnp.concatenate([gene_embed, drug_embed], axis=0)
    all_embed = jnp.pad(all_embed, ((0, NPAD - N), (0, 0)))
    tab = jnp.stack([all_embed[:, :HALF], all_embed[:, HALF:]], axis=0)
    t_all = jnp.concatenate([gene_t[:, 0], drug_t[:, 0]], axis=0)
    neg0 = neg_items[:, 0]

    mesh = plsc.VectorSubcoreMesh(core_axis_name="c", subcore_axis_name="s",
                                  num_cores=NC, num_subcores=NS)
    f32 = jnp.float32
    sc = pl.kernel(
        functools.partial(_sc_body, epad),
        out_type=(
            jax.ShapeDtypeStruct((NC, HOPS + 1, B, HALF), f32),
            jax.ShapeDtypeStruct((NC, HOPS + 1, B, HALF), f32),
            jax.ShapeDtypeStruct((NC, HOPS + 1, B, HALF), f32),
            jax.ShapeDtypeStruct((NC, NPAD, HALF), f32),
        ),
        mesh=mesh,
        compiler_params=pltpu.CompilerParams(use_tc_tiling_on_sc=False),
        scratch_types=[
            pltpu.VMEM((SROWS, CHUNK), jnp.int32),
            pltpu.VMEM((SROWS, CHUNK), jnp.int32),
            pltpu.VMEM((SROWS, CHUNK), f32),
            pltpu.VMEM((CHUNK, HALF), f32),
            pltpu.VMEM((CHUNK, HALF), f32),
            pltpu.VMEM((CHUNK, HALF), f32),
            pltpu.VMEM((CHUNK, HALF), f32),
            pltpu.VMEM((SAMP,), jnp.int32),
            pltpu.VMEM((SAMP,), jnp.int32),
            pltpu.VMEM((SAMP,), jnp.int32),
            pltpu.VMEM((SAMP,), f32),
            pltpu.VMEM((SAMP,), f32),
            pltpu.VMEM((SAMP,), f32),
            pltpu.VMEM((SAMP,), f32),
            pltpu.VMEM_SHARED((NPAD, HALF), f32),
        ] + [pltpu.SemaphoreType.DMA] * 9,
    )
    u8, p8, n8, _ = sc(tab, dst, src, val, t_all, genes, pos_items, neg0)

    loss = pl.pallas_call(
        _loss_body,
        out_shape=jax.ShapeDtypeStruct((8, 128), f32),
    )(u8, p8, n8)
    return loss[0, 0]


# speedup vs baseline: 1.5315x; 1.0314x over previous
import functools
import math

import jax
import jax.numpy as jnp
from jax import lax
from jax.experimental import pallas as pl
from jax.experimental.pallas import tpu as pltpu
from jax.experimental.pallas import tpu_sc as plsc

N_GENES = 40000
N_DRUGS = 10000
N = N_GENES + N_DRUGS
D = 64
HALF = 32
HOPS = 3
B = 4096
DECAY = 1e-4

NC = 2
NS = 16
CHUNK = 64
SUPER = 1024
SROWS = SUPER // CHUNK
NBUF = 8
SAMP = B // NS
NPAD = 50176
ROWS_PT = NPAD // NS
CP = 56


def _sc_body(epad, tab_h, dst_h, src_h, val_h, tall_h, g_h, p_h, n_h,
             u_o, po_o, ne_o, s_o,
             dstb, srcb, valb, rows, rows1, rows2, rows3, rows4, rows5,
             rows6, rows7, gidx, pidx, nidx,
             gt, pt, nt, wbuf, acc_sh, sem, semg0, semg1, semg2, semg3,
             semg4, semg5, semg6, semg7,
             sems0, sems1, sems2, sems3, sems4, sems5, sems6, sems7):
    c = lax.axis_index("c")
    s = lax.axis_index("s")
    sbase = s * SAMP
    ept = epad // NS
    n_super = ept // SUPER
    erow0 = s * (ept // CHUNK)

    kinds = ((gidx, gt, u_o, g_h, 0), (pidx, pt, po_o, p_h, N_GENES),
             (nidx, nt, ne_o, n_h, N_GENES))

    for idx, tv, _, src_h_k, off in kinds:
        pltpu.sync_copy(src_h_k.at[pl.ds(sbase, SAMP)], idx)
        if off:
            @plsc.parallel_loop(0, SAMP, 16)
            def _(i):
                idx[pl.ds(i, 16)] = idx[pl.ds(i, 16)] + off
        for j in range(SAMP // CHUNK):
            pltpu.async_copy(
                tall_h.at[idx.at[pl.ds(j * CHUNK, CHUNK)]],
                tv.at[pl.ds(j * CHUNK, CHUNK)], sem).wait()

    def samp_accum(k):
        fk = 1.0 / float(math.factorial(k))
        src_tab = tab_h if k == 0 else s_o
        for idx, tv, out, _, _ in kinds:
            @plsc.parallel_loop(0, SAMP, 16)
            def _(i):
                t = tv[pl.ds(i, 16)]
                tk = jnp.full((16,), fk, jnp.float32)
                for _ in range(k):
                    tk = tk * t
                wbuf[pl.ds(i, 16)] = jnp.exp(-t) * tk

            def sjloop(j, scarry):
                pltpu.async_copy(
                    src_tab.at[c].at[idx.at[pl.ds(j * CHUNK, CHUNK)]],
                    rows1, sem)
                pltpu.make_async_copy(
                    src_tab.at[c].at[idx.at[pl.ds(j * CHUNK, CHUNK)]],
                    rows1, sem).wait()

                @plsc.parallel_loop(0, CHUNK, 16)
                def _(i):
                    wv = wbuf[pl.ds(j * CHUNK + i, 16)]
                    for l in range(16):
                        w = jnp.full((16,), wv[l], jnp.float32)
                        rows1[i + l, pl.ds(0, 16)] = (
                            rows1[i + l, pl.ds(0, 16)] * w)
                        rows1[i + l, pl.ds(16, 16)] = (
                            rows1[i + l, pl.ds(16, 16)] * w)

                pltpu.sync_copy(
                    rows1, out.at[c, k, pl.ds(sbase + j * CHUNK, CHUNK)])
                return scarry
            lax.fori_loop(0, SAMP // CHUNK, sjloop, 0)

    def hop(first, acc_sh):
        src_tab = tab_h if first else s_o

        @plsc.parallel_loop(0, CHUNK, 1, unroll=8)
        def _(i):
            z = jnp.zeros((16,), jnp.float32)
            rows[i, pl.ds(0, 16)] = z
            rows[i, pl.ds(16, 16)] = z

        rbase = s * ROWS_PT

        def zcopy(j, carry):
            pltpu.async_copy(rows.at[pl.ds(0, CP)],
                             acc_sh.at[pl.ds(rbase + j * CP, CP)], sem)
            return carry
        lax.fori_loop(0, ROWS_PT // CP, zcopy, 0)

        def zdrain(j, carry):
            pltpu.make_async_copy(rows.at[pl.ds(0, CP)],
                                  acc_sh.at[pl.ds(rbase + j * CP, CP)],
                                  sem).wait()
            return carry
        lax.fori_loop(0, ROWS_PT // CP, zdrain, 0)
        plsc.subcore_barrier()

        bufs = (rows, rows1, rows2, rows3, rows4, rows5, rows6, rows7)
        sems = (semg0, semg1, semg2, semg3, semg4, semg5, semg6, semg7)
        ssems = (sems0, sems1, sems2, sems3, sems4, sems5, sems6, sems7)

        def echunk(sc_i, carry):
            brow = erow0 + sc_i * SROWS
            pltpu.async_copy(dst_h.at[pl.ds(brow, SROWS)], dstb, sem)
            pltpu.async_copy(src_h.at[pl.ds(brow, SROWS)], srcb, sem)
            pltpu.make_async_copy(dst_h.at[pl.ds(brow, SROWS)], dstb, sem).wait()
            pltpu.make_async_copy(src_h.at[pl.ds(brow, SROWS)], srcb, sem).wait()
            pltpu.sync_copy(val_h.at[pl.ds(brow, SROWS)], valb)
            for q in range(NBUF - 1):
                pltpu.async_copy(src_tab.at[c].at[srcb.at[q]], bufs[q], sems[q])

            def ring(rj, pcarry):
                for par in range(NBUF):
                    j = rj * NBUF + par
                    buf, bsem, bss = bufs[par], sems[par], ssems[par]
                    f = j + NBUF - 1
                    q = (par + NBUF - 1) % NBUF
                    fbuf, fsem, fss = bufs[q], sems[q], ssems[q]

                    pltpu.make_async_copy(
                        src_tab.at[c].at[srcb.at[j]], buf, bsem).wait()

                    @plsc.parallel_loop(0, CHUNK, 16)
                    def _(e):
                        vv = valb[j, pl.ds(e, 16)]
                        for l in range(16):
                            w = jnp.full((16,), vv[l], jnp.float32)
                            buf[e + l, pl.ds(0, 16)] = (
                                buf[e + l, pl.ds(0, 16)] * w)
                            buf[e + l, pl.ds(16, 16)] = (
                                buf[e + l, pl.ds(16, 16)] * w)

                    if par == 0:
                        @pl.when(j >= 1)
                        def _():
                            pltpu.make_async_copy(
                                fbuf, acc_sh.at[dstb.at[j - 1]], fss).wait()
                        pltpu.async_copy(
                            src_tab.at[c].at[srcb.at[f]], fbuf, fsem)
                    else:
                        @pl.when(f < SROWS)
                        def _():
                            pltpu.make_async_copy(
                                fbuf, acc_sh.at[dstb.at[j - 1]], fss).wait()
                            pltpu.async_copy(
                                src_tab.at[c].at[srcb.at[f]], fbuf, fsem)

                    pltpu.async_copy(buf, acc_sh.at[dstb.at[j]], bss, add=True)
                return pcarry
            lax.fori_loop(0, SROWS // NBUF, ring, 0)
            for j in range(SROWS - NBUF, SROWS):
                pltpu.make_async_copy(
                    bufs[j % NBUF], acc_sh.at[dstb.at[j]],
                    ssems[j % NBUF]).wait()
            return carry
        lax.fori_loop(0, n_super, echunk, 0)
        plsc.subcore_barrier()

        def pcopy(j, carry):
            pltpu.async_copy(acc_sh.at[pl.ds(rbase + j * CP, CP)],
                             s_o.at[c, pl.ds(rbase + j * CP, CP)], sem)
            return carry
        lax.fori_loop(0, ROWS_PT // CP, pcopy, 0)

        def pdrain(j, carry):
            pltpu.make_async_copy(acc_sh.at[pl.ds(rbase + j * CP, CP)],
                                  s_o.at[c, pl.ds(rbase + j * CP, CP)],
                                  sem).wait()
            return carry
        lax.fori_loop(0, ROWS_PT // CP, pdrain, 0)
        plsc.subcore_barrier()

    samp_accum(0)
    for k in range(1, HOPS + 1):
        hop(k == 1, acc_sh)
        samp_accum(k)


def _loss_body(u_ref, p_ref, n_ref, o_ref):
    u = jnp.sum(u_ref[...], axis=1)
    p = jnp.sum(p_ref[...], axis=1)
    n = jnp.sum(n_ref[...], axis=1)
    scale = 1.0 / float(HOPS + 1)
    s2 = scale * scale
    ps = jnp.sum(jnp.sum(u * p, axis=-1), axis=0) * s2
    ns = jnp.sum(jnp.sum(u * n, axis=-1), axis=0) * s2
    mf = jnp.mean(jnp.log(1.0 + jnp.exp(ns - ps)))
    sq = (jnp.sum(u * u) + jnp.sum(p * p) + jnp.sum(n * n)) * s2
    loss = mf + DECAY * (sq * 0.5) / float(B)
    o_ref[...] = jnp.full((8, 128), loss, jnp.float32)


def kernel(genes, pos_items, neg_items, gene_embed, drug_embed, gene_t,
           drug_t, adj_indices, adj_values):
    E = adj_values.shape[0]
    blk = NS * SUPER
    epad = ((E + blk - 1) // blk) * blk
    dst = jnp.pad(adj_indices[0], (0, epad - E)).reshape(-1, CHUNK)
    src = jnp.pad(adj_indices[1], (0, epad - E)).reshape(-1, CHUNK)
    val = jnp.pad(adj_values, (0, epad - E)).reshape(-1, CHUNK)

    all_embed = jnp.concatenate([gene_embed, drug_embed], axis=0)
    all_embed = jnp.pad(all_embed, ((0, NPAD - N), (0, 0)))
    tab = jnp.stack([all_embed[:, :HALF], all_embed[:, HALF:]], axis=0)
    t_all = jnp.concatenate([gene_t[:, 0], drug_t[:, 0]], axis=0)
    neg0 = neg_items[:, 0]

    mesh = plsc.VectorSubcoreMesh(core_axis_name="c", subcore_axis_name="s",
                                  num_cores=NC, num_subcores=NS)
    f32 = jnp.float32
    sc = pl.kernel(
        functools.partial(_sc_body, epad),
        out_type=(
            jax.ShapeDtypeStruct((NC, HOPS + 1, B, HALF), f32),
            jax.ShapeDtypeStruct((NC, HOPS + 1, B, HALF), f32),
            jax.ShapeDtypeStruct((NC, HOPS + 1, B, HALF), f32),
            jax.ShapeDtypeStruct((NC, NPAD, HALF), f32),
        ),
        mesh=mesh,
        compiler_params=pltpu.CompilerParams(use_tc_tiling_on_sc=False),
        scratch_types=[
            pltpu.VMEM((SROWS, CHUNK), jnp.int32),
            pltpu.VMEM((SROWS, CHUNK), jnp.int32),
            pltpu.VMEM((SROWS, CHUNK), f32),
            pltpu.VMEM((CHUNK, HALF), f32),
            pltpu.VMEM((CHUNK, HALF), f32),
            pltpu.VMEM((CHUNK, HALF), f32),
            pltpu.VMEM((CHUNK, HALF), f32),
            pltpu.VMEM((CHUNK, HALF), f32),
            pltpu.VMEM((CHUNK, HALF), f32),
            pltpu.VMEM((CHUNK, HALF), f32),
            pltpu.VMEM((CHUNK, HALF), f32),
            pltpu.VMEM((SAMP,), jnp.int32),
            pltpu.VMEM((SAMP,), jnp.int32),
            pltpu.VMEM((SAMP,), jnp.int32),
            pltpu.VMEM((SAMP,), f32),
            pltpu.VMEM((SAMP,), f32),
            pltpu.VMEM((SAMP,), f32),
            pltpu.VMEM((SAMP,), f32),
            pltpu.VMEM_SHARED((NPAD, HALF), f32),
        ] + [pltpu.SemaphoreType.DMA] * 17,
    )
    u8, p8, n8, _ = sc(tab, dst, src, val, t_all, genes, pos_items, neg0)

    loss = pl.pallas_call(
        _loss_body,
        out_shape=jax.ShapeDtypeStruct((8, 128), f32),
    )(u8, p8, n8)
    return loss[0, 0]
